# baseline (speedup 1.0000x reference)
import functools

import jax
import jax.numpy as jnp
from jax import lax
from jax.experimental import pallas as pl
from jax.experimental.pallas import tpu as pltpu
from jax.experimental.pallas import tpu_sc as plsc

E = 16
TOPK = 2
H = 1024
FF = 3584
T = 64

BF = 896
NF = FF // BF

NW = 32
TPW = T // NW


def _router_sc(x_hbm, gwt_hbm, cw_hbm, x_v, gwt_v, cw_v):
    wid = lax.axis_index("s") * 2 + lax.axis_index("c")
    base = wid * TPW
    pltpu.sync_copy(x_hbm.at[pl.ds(base * H, TPW * H)], x_v)
    pltpu.sync_copy(gwt_hbm, gwt_v)
    for tl in range(TPW):
        def body(c, acc):
            xc = x_v[pl.ds(tl * H + c * 16, 16)]
            for j in range(16):
                acc = acc + xc[j] * gwt_v[pl.ds((c * 16 + j) * E, E)]
            return acc
        logits = lax.fori_loop(0, H // 16, body, jnp.zeros((E,), jnp.float32))
        idx = lax.iota(jnp.int32, E)

        dnums = lax.GatherDimensionNumbers(
            offset_dims=(), collapsed_slice_dims=(0,), start_index_map=(0,))

        def _perm(v, ix):
            return lax.gather(
                v, ix[:, None], dnums, slice_sizes=(1,),
                mode=lax.GatherScatterMode.PROMISE_IN_BOUNDS)

        def _bfly(v, op):
            for k in (8, 4, 2, 1):
                v = op(v, _perm(v, idx ^ k))
            return v

        p = jnp.exp(logits - _bfly(logits, jnp.maximum))
        p = p / _bfly(p, jnp.add)
        m1 = _bfly(p, jnp.maximum)
        i1 = _bfly(jnp.where(p == m1, idx, E), jnp.minimum)
        mask1 = idx == i1
        p2 = jnp.where(mask1, -1.0, p)
        m2 = _bfly(p2, jnp.maximum)
        i2 = _bfly(jnp.where(p2 == m2, idx, E), jnp.minimum)
        mask2 = idx == i2
        cw = (jnp.where(mask1, m1, 0.0) + jnp.where(mask2, m2, 0.0)) / (m1 + m2)
        cw_v[pl.ds(tl * E, E)] = cw
    pltpu.sync_copy(cw_v, cw_hbm.at[pl.ds(base * E, TPW * E)])


_router = functools.partial(
    pl.kernel,
    mesh=plsc.VectorSubcoreMesh(core_axis_name="c", subcore_axis_name="s"),
    out_type=jax.ShapeDtypeStruct((T * E,), jnp.float32),
    scratch_types=[
        pltpu.VMEM((TPW * H,), jnp.float32),
        pltpu.VMEM((H * E,), jnp.float32),
        pltpu.VMEM((TPW * E,), jnp.float32),
    ],
)(_router_sc)


def _moe_body(cw_ref, x_ref, w1_ref, w2_ref, w3_ref, out_ref):
    e = pl.program_id(0)
    f = pl.program_id(1)

    @pl.when((e == 0) & (f == 0))
    def _init():
        out_ref[...] = jnp.zeros_like(out_ref)

    x = x_ref[...]
    w1b = w1_ref[0]
    w3b = w3_ref[0]
    w2b = w2_ref[0]
    gate = lax.dot_general(
        x, w1b, (((1,), (1,)), ((), ())), preferred_element_type=jnp.float32)
    up = lax.dot_general(
        x, w3b, (((1,), (1,)), ((), ())), preferred_element_type=jnp.float32)
    inter = gate * lax.logistic(gate) * up
    partial = lax.dot_general(
        inter, w2b, (((1,), (1,)), ((), ())), preferred_element_type=jnp.float32)
    lanes = lax.broadcasted_iota(jnp.int32, (T, E), 1)
    cw_col = jnp.sum(jnp.where(lanes == e, cw_ref[...], 0.0),
                     axis=-1, keepdims=True)
    out_ref[...] += cw_col * partial


@jax.jit
def kernel(x, gate_w, w1, w2, w3):
    cw = _router(x.reshape(-1), gate_w.T.reshape(-1)).reshape(T, E)
    return pl.pallas_call(
        _moe_body,
        grid=(E, NF),
        in_specs=[
            pl.BlockSpec((T, E), lambda e, f: (0, 0)),
            pl.BlockSpec((T, H), lambda e, f: (0, 0)),
            pl.BlockSpec((1, BF, H), lambda e, f: (e, f, 0)),
            pl.BlockSpec((1, H, BF), lambda e, f: (e, 0, f)),
            pl.BlockSpec((1, BF, H), lambda e, f: (e, f, 0)),
        ],
        out_specs=pl.BlockSpec((T, H), lambda e, f: (0, 0)),
        out_shape=jax.ShapeDtypeStruct((T, H), jnp.float32),
        compiler_params=pltpu.CompilerParams(
            dimension_semantics=("arbitrary", "arbitrary"),
        ),
    )(cw, x, w1, w2, w3)


# speedup vs baseline: 1.0081x; 1.0081x over previous
import functools

import jax
import jax.numpy as jnp
from jax import lax
from jax.experimental import pallas as pl
from jax.experimental.pallas import tpu as pltpu
from jax.experimental.pallas import tpu_sc as plsc

E = 16
TOPK = 2
H = 1024
FF = 3584
T = 64

BF = 896
NF = FF // BF

NW = 32
TPW = T // NW


def _router_sc(x_hbm, gwt_hbm, cw_hbm, x_v, gwt_v, cw_v):
    wid = lax.axis_index("s") * 2 + lax.axis_index("c")
    base = wid * TPW
    pltpu.sync_copy(x_hbm.at[pl.ds(base * H, TPW * H)], x_v)
    pltpu.sync_copy(gwt_hbm, gwt_v)
    for tl in range(TPW):
        def body(c, acc):
            xc = x_v[pl.ds(tl * H + c * 16, 16)]
            for j in range(16):
                acc = acc + xc[j] * gwt_v[pl.ds((c * 16 + j) * E, E)]
            return acc
        logits = lax.fori_loop(0, H // 16, body, jnp.zeros((E,), jnp.float32))
        idx = lax.iota(jnp.int32, E)

        dnums = lax.GatherDimensionNumbers(
            offset_dims=(), collapsed_slice_dims=(0,), start_index_map=(0,))

        def _perm(v, ix):
            return lax.gather(
                v, ix[:, None], dnums, slice_sizes=(1,),
                mode=lax.GatherScatterMode.PROMISE_IN_BOUNDS)

        def _bfly(v, op):
            for k in (8, 4, 2, 1):
                v = op(v, _perm(v, idx ^ k))
            return v

        p = jnp.exp(logits - _bfly(logits, jnp.maximum))
        p = p / _bfly(p, jnp.add)
        m1 = _bfly(p, jnp.maximum)
        i1 = _bfly(jnp.where(p == m1, idx, E), jnp.minimum)
        mask1 = idx == i1
        p2 = jnp.where(mask1, -1.0, p)
        m2 = _bfly(p2, jnp.maximum)
        i2 = _bfly(jnp.where(p2 == m2, idx, E), jnp.minimum)
        mask2 = idx == i2
        cw = (jnp.where(mask1, m1, 0.0) + jnp.where(mask2, m2, 0.0)) / (m1 + m2)
        cw_v[pl.ds(tl * E, E)] = cw
    pltpu.sync_copy(cw_v, cw_hbm.at[pl.ds(base * E, TPW * E)])


_router = functools.partial(
    pl.kernel,
    mesh=plsc.VectorSubcoreMesh(core_axis_name="c", subcore_axis_name="s"),
    out_type=jax.ShapeDtypeStruct((T * E,), jnp.float32),
    scratch_types=[
        pltpu.VMEM((TPW * H,), jnp.float32),
        pltpu.VMEM((H * E,), jnp.float32),
        pltpu.VMEM((TPW * E,), jnp.float32),
    ],
)(_router_sc)


def _moe_body(x_ref, w1_ref, w2_ref, w3_ref, y_ref):
    f = pl.program_id(1)

    @pl.when(f == 0)
    def _init():
        y_ref[...] = jnp.zeros_like(y_ref)

    x = x_ref[...]
    w1b = w1_ref[0]
    w3b = w3_ref[0]
    w2b = w2_ref[0]
    gate = lax.dot_general(
        x, w1b, (((1,), (1,)), ((), ())), preferred_element_type=jnp.float32)
    up = lax.dot_general(
        x, w3b, (((1,), (1,)), ((), ())), preferred_element_type=jnp.float32)
    inter = gate * lax.logistic(gate) * up
    partial = lax.dot_general(
        inter, w2b, (((1,), (1,)), ((), ())), preferred_element_type=jnp.float32)
    y_ref[0] += partial


def _combine_body(cw_ref, y_ref, out_ref):
    cw = cw_ref[...]
    acc = cw[:, 0:1] * y_ref[0]
    for e in range(1, E):
        acc += cw[:, e:e + 1] * y_ref[e]
    out_ref[...] = acc


@jax.jit
def kernel(x, gate_w, w1, w2, w3):
    cw = _router(x.reshape(-1), gate_w.T.reshape(-1)).reshape(T, E)
    y = pl.pallas_call(
        _moe_body,
        grid=(E, NF),
        in_specs=[
            pl.BlockSpec((T, H), lambda e, f: (0, 0)),
            pl.BlockSpec((1, BF, H), lambda e, f: (e, f, 0)),
            pl.BlockSpec((1, H, BF), lambda e, f: (e, 0, f)),
            pl.BlockSpec((1, BF, H), lambda e, f: (e, f, 0)),
        ],
        out_specs=pl.BlockSpec((1, T, H), lambda e, f: (e, 0, 0)),
        out_shape=jax.ShapeDtypeStruct((E, T, H), jnp.float32),
        compiler_params=pltpu.CompilerParams(
            dimension_semantics=("arbitrary", "arbitrary"),
        ),
    )(x, w1, w2, w3)
    return pl.pallas_call(
        _combine_body,
        in_specs=[
            pl.BlockSpec((T, E), lambda: (0, 0)),
            pl.BlockSpec((E, T, H), lambda: (0, 0, 0)),
        ],
        out_specs=pl.BlockSpec((T, H), lambda: (0, 0)),
        out_shape=jax.ShapeDtypeStruct((T, H), jnp.float32),
    )(cw, y)


# speedup vs baseline: 1.1300x; 1.1208x over previous
import functools

import jax
import jax.numpy as jnp
from jax.experimental import pallas as pl
from jax.experimental.pallas import tpu as pltpu

E = 16
TOPK = 2
H = 1024
FF = 3584
T = 64

BF = 896
NF = FF // BF


def _moe_body(x_ref, gate_w_ref, w1_ref, w2_ref, w3_ref, out_ref, cw_ref):
    e = pl.program_id(0)
    f = pl.program_id(1)

    @pl.when((e == 0) & (f == 0))
    def _routing():
        xb = x_ref[...].astype(jnp.bfloat16)
        gb = gate_w_ref[...].astype(jnp.bfloat16)
        logits = jax.lax.dot_general(
            xb, gb, (((1,), (1,)), ((), ())),
            preferred_element_type=jnp.float32)
        p = jax.nn.softmax(logits, axis=-1)
        idx = jax.lax.broadcasted_iota(jnp.int32, (T, E), 1)
        m1 = jnp.max(p, axis=-1, keepdims=True)
        i1 = jnp.min(jnp.where(p == m1, idx, E), axis=-1, keepdims=True)
        mask1 = idx == i1
        p2 = jnp.where(mask1, -1.0, p)
        m2 = jnp.max(p2, axis=-1, keepdims=True)
        i2 = jnp.min(jnp.where(p2 == m2, idx, E), axis=-1, keepdims=True)
        mask2 = idx == i2
        s = m1 + m2
        cw = (jnp.where(mask1, m1, 0.0) + jnp.where(mask2, m2, 0.0)) / s
        cw_ref[:, 0:E] = cw
        out_ref[...] = jnp.zeros_like(out_ref)

    x = x_ref[...]
    w1b = w1_ref[0]
    w3b = w3_ref[0]
    w2b = w2_ref[0]
    gate = jax.lax.dot_general(
        x, w1b, (((1,), (1,)), ((), ())), preferred_element_type=jnp.float32)
    up = jax.lax.dot_general(
        x, w3b, (((1,), (1,)), ((), ())), preferred_element_type=jnp.float32)
    inter = gate * jax.lax.logistic(gate) * up
    partial = jax.lax.dot_general(
        inter, w2b, (((1,), (1,)), ((), ())), preferred_element_type=jnp.float32)
    lanes = jax.lax.broadcasted_iota(jnp.int32, (T, 128), 1)
    cw_col = jnp.sum(jnp.where(lanes == e, cw_ref[...], 0.0),
                     axis=-1, keepdims=True)
    out_ref[...] += cw_col * partial


@jax.jit
def kernel(x, gate_w, w1, w2, w3):
    return pl.pallas_call(
        _moe_body,
        grid=(E, NF),
        in_specs=[
            pl.BlockSpec((T, H), lambda e, f: (0, 0)),
            pl.BlockSpec((E, H), lambda e, f: (0, 0)),
            pl.BlockSpec((1, BF, H), lambda e, f: (e, f, 0)),
            pl.BlockSpec((1, H, BF), lambda e, f: (e, 0, f)),
            pl.BlockSpec((1, BF, H), lambda e, f: (e, f, 0)),
        ],
        out_specs=pl.BlockSpec((T, H), lambda e, f: (0, 0)),
        out_shape=jax.ShapeDtypeStruct((T, H), jnp.float32),
        scratch_shapes=[pltpu.VMEM((T, 128), jnp.float32)],
        compiler_params=pltpu.CompilerParams(
            dimension_semantics=("arbitrary", "arbitrary"),
        ),
    )(x, gate_w, w1, w2, w3)
